# baseline (speedup 1.0000x reference)
import functools

import jax
import jax.numpy as jnp
from jax import lax
from jax.experimental import pallas as pl
from jax.experimental.pallas import tpu as pltpu
from jax.experimental.pallas import tpu_sc as plsc

CHUNK = 128


def kernel(x, embedding):
    B, L = x.shape
    V, D = embedding.shape
    N = B * L
    info = plsc.get_sparse_core_info()
    NC, NS, LN = info.num_cores, info.num_subcores, info.num_lanes
    NW = NC * NS
    n_chunks = N // CHUNK
    chunks_per_w = n_chunks // NW
    per_w = chunks_per_w * CHUNK

    idx_flat = x.T.reshape(N)
    table2 = embedding.reshape(V // 2, 2 * D)

    mesh = plsc.VectorSubcoreMesh(core_axis_name="c", subcore_axis_name="s")

    @functools.partial(
        pl.kernel,
        mesh=mesh,
        out_type=jax.ShapeDtypeStruct((N, D), jnp.float32),
        compiler_params=pltpu.CompilerParams(use_tc_tiling_on_sc=False),
        scratch_types=[
            pltpu.VMEM((per_w,), jnp.int32),
            pltpu.VMEM((per_w,), jnp.int32),
            pltpu.VMEM((2, CHUNK, 2 * D), jnp.float32),
            pltpu.VMEM((CHUNK, D), jnp.float32),
            pltpu.SemaphoreType.DMA,
        ],
    )
    def gather_k(idx_hbm, table_hbm, out_hbm, idx_v, idxp_v, rows_v, obuf_v, sem):
        wid = lax.axis_index("s") * NC + lax.axis_index("c")
        base = wid * per_w
        pltpu.sync_copy(idx_hbm.at[pl.ds(base, per_w)], idx_v)

        def mk_pairs(g, carry):
            v = idx_v[pl.ds(g * LN, LN)]
            idxp_v[pl.ds(g * LN, LN)] = lax.shift_right_logical(v, 1)
            return carry

        lax.fori_loop(0, per_w // LN, mk_pairs, 0)

        def start_gather(j, slot):
            return pltpu.async_copy(
                table_hbm.at[idxp_v.at[pl.ds(j * CHUNK, CHUNK)]],
                rows_v.at[slot],
                sem,
            )

        start_gather(0, 0).wait()

        def half_body(j, slot):
            @pl.when(j + 1 < chunks_per_w)
            def _():
                start_gather(j + 1, 1 - slot)

            def bgroup(g, carry2):
                hv = idx_v[pl.ds(j * CHUNK + g * LN, LN)] & 1
                for bp in range(LN):
                    b = g * LN + bp
                    off = hv[bp] * D
                    for t in range(D // LN):
                        obuf_v[b, pl.ds(t * LN, LN)] = rows_v[
                            slot, b, pl.ds(off + t * LN, LN)
                        ]
                return carry2

            lax.fori_loop(0, CHUNK // LN, bgroup, 0)

            @pl.when(j + 1 < chunks_per_w)
            def _():
                pltpu.make_async_copy(
                    table_hbm.at[idxp_v.at[pl.ds((j + 1) * CHUNK, CHUNK)]],
                    rows_v.at[1 - slot],
                    sem,
                ).wait()

            pltpu.sync_copy(obuf_v, out_hbm.at[pl.ds(base + j * CHUNK, CHUNK)])

        def body(j2, carry):
            half_body(2 * j2, 0)
            half_body(2 * j2 + 1, 1)
            return carry

        lax.fori_loop(0, chunks_per_w // 2, body, 0)

    out = gather_k(idx_flat, table2)
    return out.reshape(L, B, D).swapaxes(0, 1)


# speedup vs baseline: 1.1325x; 1.1325x over previous
import functools

import jax
import jax.numpy as jnp
from jax import lax
from jax.experimental import pallas as pl
from jax.experimental.pallas import tpu as pltpu
from jax.experimental.pallas import tpu_sc as plsc

CHUNK = 128


def kernel(x, embedding):
    B, L = x.shape
    V, D = embedding.shape
    N = B * L
    info = plsc.get_sparse_core_info()
    NC, NS = info.num_cores, info.num_subcores
    NW = NC * NS
    n_chunks = N // CHUNK
    chunks_per_w = n_chunks // NW
    per_w = chunks_per_w * CHUNK

    idx_flat = x.T.reshape(N)
    table_p = jnp.pad(embedding, ((0, 0), (0, 2 * D - D)))

    mesh = plsc.VectorSubcoreMesh(core_axis_name="c", subcore_axis_name="s")

    @functools.partial(
        pl.kernel,
        mesh=mesh,
        out_type=jax.ShapeDtypeStruct((N, D), jnp.float32),
        compiler_params=pltpu.CompilerParams(use_tc_tiling_on_sc=False),
        scratch_types=[
            pltpu.VMEM((per_w,), jnp.int32),
            pltpu.VMEM((2, CHUNK, 2 * D), jnp.float32),
            pltpu.SemaphoreType.DMA,
        ],
    )
    def gather_k(idx_hbm, table_hbm, out_hbm, idx_v, rows_v, sem):
        wid = lax.axis_index("s") * NC + lax.axis_index("c")
        base = wid * per_w
        pltpu.sync_copy(idx_hbm.at[pl.ds(base, per_w)], idx_v)

        def start_gather(j, slot):
            return pltpu.async_copy(
                table_hbm.at[idx_v.at[pl.ds(j * CHUNK, CHUNK)]],
                rows_v.at[slot],
                sem,
            )

        start_gather(0, 0).wait()

        def half_body(j, slot):
            @pl.when(j + 1 < chunks_per_w)
            def _():
                start_gather(j + 1, 1 - slot)

            pltpu.sync_copy(
                rows_v.at[slot, :, pl.ds(0, D)],
                out_hbm.at[pl.ds(base + j * CHUNK, CHUNK)],
            )

            @pl.when(j + 1 < chunks_per_w)
            def _():
                pltpu.make_async_copy(
                    table_hbm.at[idx_v.at[pl.ds((j + 1) * CHUNK, CHUNK)]],
                    rows_v.at[1 - slot],
                    sem,
                ).wait()

        def body(j2, carry):
            half_body(2 * j2, 0)
            half_body(2 * j2 + 1, 1)
            return carry

        lax.fori_loop(0, chunks_per_w // 2, body, 0)

    out = gather_k(idx_flat, table_p)
    return out.reshape(L, B, D).swapaxes(0, 1)


# speedup vs baseline: 1.2384x; 1.0935x over previous
import functools

import jax
import jax.numpy as jnp
from jax import lax
from jax.experimental import pallas as pl
from jax.experimental.pallas import tpu as pltpu
from jax.experimental.pallas import tpu_sc as plsc

CHUNK = 128


def kernel(x, embedding):
    B, L = x.shape
    V, D = embedding.shape
    N = B * L
    info = plsc.get_sparse_core_info()
    NC, NS = info.num_cores, info.num_subcores
    NW = NC * NS
    n_chunks = N // CHUNK
    chunks_per_w = n_chunks // NW
    per_w = chunks_per_w * CHUNK

    idx_flat = x.T.reshape(N)
    table_p = jnp.pad(embedding, ((0, 0), (0, 2 * D - D)))

    mesh = plsc.VectorSubcoreMesh(core_axis_name="c", subcore_axis_name="s")

    @functools.partial(
        pl.kernel,
        mesh=mesh,
        out_type=jax.ShapeDtypeStruct((N, 2 * D), jnp.float32),
        compiler_params=pltpu.CompilerParams(use_tc_tiling_on_sc=False),
        scratch_types=[
            pltpu.VMEM((per_w,), jnp.int32),
            pltpu.VMEM((2, CHUNK, 2 * D), jnp.float32),
            pltpu.SemaphoreType.DMA,
        ],
    )
    def gather_k(idx_hbm, table_hbm, out_hbm, idx_v, rows_v, sem):
        wid = lax.axis_index("s") * NC + lax.axis_index("c")
        base = wid * per_w
        pltpu.sync_copy(idx_hbm.at[pl.ds(base, per_w)], idx_v)

        def start_gather(j, slot):
            return pltpu.async_copy(
                table_hbm.at[idx_v.at[pl.ds(j * CHUNK, CHUNK)]],
                rows_v.at[slot],
                sem,
            )

        start_gather(0, 0).wait()

        def half_body(j, slot):
            @pl.when(j + 1 < chunks_per_w)
            def _():
                start_gather(j + 1, 1 - slot)

            pltpu.sync_copy(
                rows_v.at[slot],
                out_hbm.at[pl.ds(base + j * CHUNK, CHUNK)],
            )

            @pl.when(j + 1 < chunks_per_w)
            def _():
                pltpu.make_async_copy(
                    table_hbm.at[idx_v.at[pl.ds((j + 1) * CHUNK, CHUNK)]],
                    rows_v.at[1 - slot],
                    sem,
                ).wait()

        def body(j2, carry):
            half_body(2 * j2, 0)
            half_body(2 * j2 + 1, 1)
            return carry

        lax.fori_loop(0, chunks_per_w // 2, body, 0)

    out = gather_k(idx_flat, table_p)
    return out.reshape(L, B, 2 * D)[:, :, :D].swapaxes(0, 1)
